# baseline (speedup 1.0000x reference)
import functools

import jax
import jax.numpy as jnp
from jax import lax
from jax.experimental import pallas as pl
from jax.experimental.pallas import tpu as pltpu
from jax.experimental.pallas import tpu_sc as plsc

N_NODES = 10000
N_EDGES = 320000
D = 128
DW = D // 2

_NC = 2
_NS = 16
_NW = _NC * _NS
_EPW = N_EDGES // _NW
_CH = 80
_NCHUNK = _EPW // _CH


def _pack_bf16_pair(m):
    lo = m[:, :DW].astype(jnp.bfloat16)
    hi = m[:, DW:].astype(jnp.bfloat16)
    lo_i = lax.bitcast_convert_type(lo, jnp.uint16).astype(jnp.int32)
    hi_i = lax.bitcast_convert_type(hi, jnp.uint16).astype(jnp.int32)
    return lo_i | (hi_i << 16)


def _proj_body(x_ref, wa_ref, wb_ref, p_ref, q_ref):
    xv = x_ref[...]
    p = jnp.dot(xv, wa_ref[...], preferred_element_type=jnp.float32)
    q = jnp.dot(xv, wb_ref[...], preferred_element_type=jnp.float32)
    p_ref[...] = _pack_bf16_pair(p)
    q_ref[...] = _pack_bf16_pair(q)


def _project_nodes(x, wa, wb):
    blk = 2000
    grid = N_NODES // blk
    return pl.pallas_call(
        _proj_body,
        grid=(grid,),
        in_specs=[
            pl.BlockSpec((blk, D), lambda i: (i, 0)),
            pl.BlockSpec((D, D), lambda i: (0, 0)),
            pl.BlockSpec((D, D), lambda i: (0, 0)),
        ],
        out_specs=[
            pl.BlockSpec((blk, DW), lambda i: (i, 0)),
            pl.BlockSpec((blk, DW), lambda i: (i, 0)),
        ],
        out_shape=[
            jax.ShapeDtypeStruct((N_NODES, DW), jnp.int32),
            jax.ShapeDtypeStruct((N_NODES, DW), jnp.int32),
        ],
    )(x, wa, wb)


def _gather_body(p_hbm, q_hbm, src_hbm, dst_hbm, gs_hbm, gd_hbm,
                 si_v, di_v, rs_v, rd_v, sem_s, sem_d):
    wid = lax.axis_index("s") * _NC + lax.axis_index("c")
    wbase = wid * _EPW

    def chunk_body(k, carry):
        base = wbase + k * _CH
        pltpu.sync_copy(src_hbm.at[pl.ds(base, _CH)], si_v)
        pltpu.sync_copy(dst_hbm.at[pl.ds(base, _CH)], di_v)
        cp_s = pltpu.async_copy(p_hbm.at[si_v], rs_v, sem_s)
        cp_d = pltpu.async_copy(q_hbm.at[di_v], rd_v, sem_d)
        cp_s.wait()
        cp_d.wait()
        pltpu.sync_copy(rs_v, gs_hbm.at[pl.ds(base, _CH)])
        pltpu.sync_copy(rd_v, gd_hbm.at[pl.ds(base, _CH)])
        return carry

    lax.fori_loop(0, _NCHUNK, chunk_body, 0)


def _gather_pairs(p, q, src, dst):
    mesh = plsc.VectorSubcoreMesh(core_axis_name="c", subcore_axis_name="s")
    fn = functools.partial(
        pl.kernel,
        mesh=mesh,
        compiler_params=pltpu.CompilerParams(use_tc_tiling_on_sc=False),
        out_type=[
            jax.ShapeDtypeStruct((N_EDGES, DW), jnp.int32),
            jax.ShapeDtypeStruct((N_EDGES, DW), jnp.int32),
        ],
        scratch_types=[
            pltpu.VMEM((_CH,), jnp.int32),
            pltpu.VMEM((_CH,), jnp.int32),
            pltpu.VMEM((_CH, DW), jnp.int32),
            pltpu.VMEM((_CH, DW), jnp.int32),
            pltpu.SemaphoreType.DMA,
            pltpu.SemaphoreType.DMA,
        ],
    )(_gather_body)
    return fn(p, q, src, dst)


def _unpack_bf16_pair(g):
    gl = lax.bitcast_convert_type(g << 16, jnp.float32)
    gh = lax.bitcast_convert_type(g & jnp.int32(-65536), jnp.float32)
    return jnp.concatenate([gl, gh], axis=1)


def _mlp_body(gs_ref, gd_ref, ea_ref, w1c_ref, b1_ref, w2_ref, b2_ref, o_ref):
    gfull = _unpack_bf16_pair(gs_ref[...]) + _unpack_bf16_pair(gd_ref[...])
    h = gfull + jnp.dot(ea_ref[...], w1c_ref[...],
                        preferred_element_type=jnp.float32) + b1_ref[...]
    h = jnp.maximum(h, 0.0)
    o_ref[...] = jnp.dot(h, w2_ref[...],
                         preferred_element_type=jnp.float32) + b2_ref[...]


def _edge_mlp(gs, gd, ea, w1c, b1, w2, b2):
    blk = 2560
    grid = N_EDGES // blk
    de = ea.shape[1]
    eo = w2.shape[1]
    return pl.pallas_call(
        _mlp_body,
        grid=(grid,),
        in_specs=[
            pl.BlockSpec((blk, DW), lambda i: (i, 0)),
            pl.BlockSpec((blk, DW), lambda i: (i, 0)),
            pl.BlockSpec((blk, de), lambda i: (i, 0)),
            pl.BlockSpec((de, D), lambda i: (0, 0)),
            pl.BlockSpec((1, D), lambda i: (0, 0)),
            pl.BlockSpec((D, eo), lambda i: (0, 0)),
            pl.BlockSpec((1, eo), lambda i: (0, 0)),
        ],
        out_specs=pl.BlockSpec((blk, eo), lambda i: (i, 0)),
        out_shape=jax.ShapeDtypeStruct((N_EDGES, eo), jnp.float32),
    )(gs, gd, ea, w1c, b1, w2, b2)


def kernel(x, edge_index, edge_attr, W1, b1, W2, b2):
    wa = W1[:D]
    wb = W1[D:2 * D]
    w1c = W1[2 * D:]
    p, q = _project_nodes(x, wa, wb)
    gs, gd = _gather_pairs(p, q, edge_index[0], edge_index[1])
    return _edge_mlp(gs, gd, edge_attr, w1c, b1.reshape(1, D), W2,
                     b2.reshape(1, -1))


# speedup vs baseline: 1.1925x; 1.1925x over previous
import functools

import jax
import jax.numpy as jnp
from jax import lax
from jax.experimental import pallas as pl
from jax.experimental.pallas import tpu as pltpu
from jax.experimental.pallas import tpu_sc as plsc

N_NODES = 10000
N_EDGES = 320000
D = 128
DW = D // 2

_NC = 2
_NS = 16
_NW = _NC * _NS
_EPW = N_EDGES // _NW
_CH = 80
_NCHUNK = _EPW // _CH


def _pack_bf16_pair(m):
    lo = m[:, :DW].astype(jnp.bfloat16)
    hi = m[:, DW:].astype(jnp.bfloat16)
    lo_i = lax.bitcast_convert_type(lo, jnp.uint16).astype(jnp.int32)
    hi_i = lax.bitcast_convert_type(hi, jnp.uint16).astype(jnp.int32)
    return lo_i | (hi_i << 16)


def _proj_body(x_ref, wa_ref, wb_ref, p_ref, q_ref):
    xv = x_ref[...]
    p = jnp.dot(xv, wa_ref[...], preferred_element_type=jnp.float32)
    q = jnp.dot(xv, wb_ref[...], preferred_element_type=jnp.float32)
    p_ref[...] = _pack_bf16_pair(p)
    q_ref[...] = _pack_bf16_pair(q)


def _project_nodes(x, wa, wb):
    blk = 2000
    grid = N_NODES // blk
    return pl.pallas_call(
        _proj_body,
        grid=(grid,),
        in_specs=[
            pl.BlockSpec((blk, D), lambda i: (i, 0)),
            pl.BlockSpec((D, D), lambda i: (0, 0)),
            pl.BlockSpec((D, D), lambda i: (0, 0)),
        ],
        out_specs=[
            pl.BlockSpec((blk, DW), lambda i: (i, 0)),
            pl.BlockSpec((blk, DW), lambda i: (i, 0)),
        ],
        out_shape=[
            jax.ShapeDtypeStruct((N_NODES, DW), jnp.int32),
            jax.ShapeDtypeStruct((N_NODES, DW), jnp.int32),
        ],
    )(x, wa, wb)


_NB = 5


def _gather_body(p_hbm, q_hbm, src_hbm, dst_hbm, gs_hbm, gd_hbm,
                 si_all, di_all, rs0, rs1, rs2, rs3, rs4,
                 rd0, rd1, rd2, rd3, rd4,
                 gsem_s, gsem_d, wsem_s, wsem_d):
    wid = lax.axis_index("s") * _NC + lax.axis_index("c")
    wbase = wid * _EPW
    rs = [rs0, rs1, rs2, rs3, rs4]
    rd = [rd0, rd1, rd2, rd3, rd4]

    def gather_descs(k, b):
        isl = pl.ds(k * _CH, _CH)
        return (
            pltpu.make_async_copy(p_hbm.at[si_all.at[isl]], rs[b], gsem_s.at[b]),
            pltpu.make_async_copy(q_hbm.at[di_all.at[isl]], rd[b], gsem_d.at[b]),
        )

    def wb_descs(k, b):
        osl = pl.ds(wbase + k * _CH, _CH)
        return (
            pltpu.make_async_copy(rs[b], gs_hbm.at[osl], wsem_s.at[b]),
            pltpu.make_async_copy(rd[b], gd_hbm.at[osl], wsem_d.at[b]),
        )

    pltpu.sync_copy(src_hbm.at[pl.ds(wbase, _EPW)], si_all)
    pltpu.sync_copy(dst_hbm.at[pl.ds(wbase, _EPW)], di_all)
    for dsc in gather_descs(0, 0):
        dsc.start()

    def outer(kk, carry):
        for b in range(_NB):
            k = kk * _NB + b
            k1 = k + 1
            nb = (b + 1) % _NB

            @pl.when(k1 < _NCHUNK)
            def _prefetch():
                @pl.when(k1 >= _NB)
                def _reclaim():
                    for dsc in wb_descs(k1 - _NB, nb):
                        dsc.wait()

                for dsc in gather_descs(k1, nb):
                    dsc.start()

            for dsc in gather_descs(k, b):
                dsc.wait()
            for dsc in wb_descs(k, b):
                dsc.start()
        return carry

    lax.fori_loop(0, _NCHUNK // _NB, outer, 0)
    for b in range(_NB):
        for dsc in wb_descs(_NCHUNK - _NB + b, b):
            dsc.wait()


def _gather_pairs(p, q, src, dst):
    mesh = plsc.VectorSubcoreMesh(core_axis_name="c", subcore_axis_name="s")
    fn = functools.partial(
        pl.kernel,
        mesh=mesh,
        compiler_params=pltpu.CompilerParams(use_tc_tiling_on_sc=False),
        out_type=[
            jax.ShapeDtypeStruct((N_EDGES, DW), jnp.int32),
            jax.ShapeDtypeStruct((N_EDGES, DW), jnp.int32),
        ],
        scratch_types=(
            [pltpu.VMEM((_EPW,), jnp.int32)] * 2
            + [pltpu.VMEM((_CH, DW), jnp.int32)] * (2 * _NB)
            + [pltpu.SemaphoreType.DMA((_NB,))] * 4
        ),
    )(_gather_body)
    return fn(p, q, src, dst)


def _unpack_bf16_pair(g):
    gl = lax.bitcast_convert_type(g << 16, jnp.float32)
    gh = lax.bitcast_convert_type(g & jnp.int32(-65536), jnp.float32)
    return jnp.concatenate([gl, gh], axis=1)


def _mlp_body(gs_ref, gd_ref, ea_ref, w1c_ref, b1_ref, w2_ref, b2_ref, o_ref):
    gfull = _unpack_bf16_pair(gs_ref[...]) + _unpack_bf16_pair(gd_ref[...])
    h = gfull + jnp.dot(ea_ref[...], w1c_ref[...],
                        preferred_element_type=jnp.float32) + b1_ref[...]
    h = jnp.maximum(h, 0.0)
    o_ref[...] = jnp.dot(h, w2_ref[...],
                         preferred_element_type=jnp.float32) + b2_ref[...]


def _edge_mlp(gs, gd, ea, w1c, b1, w2, b2):
    blk = 2560
    grid = N_EDGES // blk
    de = ea.shape[1]
    eo = w2.shape[1]
    return pl.pallas_call(
        _mlp_body,
        grid=(grid,),
        in_specs=[
            pl.BlockSpec((blk, DW), lambda i: (i, 0)),
            pl.BlockSpec((blk, DW), lambda i: (i, 0)),
            pl.BlockSpec((blk, de), lambda i: (i, 0)),
            pl.BlockSpec((de, D), lambda i: (0, 0)),
            pl.BlockSpec((1, D), lambda i: (0, 0)),
            pl.BlockSpec((D, eo), lambda i: (0, 0)),
            pl.BlockSpec((1, eo), lambda i: (0, 0)),
        ],
        out_specs=pl.BlockSpec((blk, eo), lambda i: (i, 0)),
        out_shape=jax.ShapeDtypeStruct((N_EDGES, eo), jnp.float32),
    )(gs, gd, ea, w1c, b1, w2, b2)


def kernel(x, edge_index, edge_attr, W1, b1, W2, b2):
    wa = W1[:D]
    wb = W1[D:2 * D]
    w1c = W1[2 * D:]
    p, q = _project_nodes(x, wa, wb)
    gs, gd = _gather_pairs(p, q, edge_index[0], edge_index[1])
    return _edge_mlp(gs, gd, edge_attr, w1c, b1.reshape(1, D), W2,
                     b2.reshape(1, -1))


# speedup vs baseline: 1.7051x; 1.4299x over previous
import functools

import jax
import jax.numpy as jnp
from jax import lax
from jax.experimental import pallas as pl
from jax.experimental.pallas import tpu as pltpu
from jax.experimental.pallas import tpu_sc as plsc

N_NODES = 10000
N_EDGES = 320000
D = 128

_NC = 2
_NS = 16
_NW = _NC * _NS
_EPW = N_EDGES // _NW
_CH = 80
_NCHUNK = _EPW // _CH
_NB = 5


def _proj_body(x_ref, wa_ref, wb_ref, p_ref, q_ref):
    xv = x_ref[...]
    p_ref[...] = jnp.dot(xv, wa_ref[...], preferred_element_type=jnp.float32)
    q_ref[...] = jnp.dot(xv, wb_ref[...], preferred_element_type=jnp.float32)


def _project_nodes(x, wa, wb):
    blk = 2000
    grid = N_NODES // blk
    return pl.pallas_call(
        _proj_body,
        grid=(grid,),
        in_specs=[
            pl.BlockSpec((blk, D), lambda i: (i, 0)),
            pl.BlockSpec((D, D), lambda i: (0, 0)),
            pl.BlockSpec((D, D), lambda i: (0, 0)),
        ],
        out_specs=[
            pl.BlockSpec((blk, D), lambda i: (i, 0)),
            pl.BlockSpec((blk, D), lambda i: (i, 0)),
        ],
        out_shape=[
            jax.ShapeDtypeStruct((N_NODES, D), jnp.float32),
            jax.ShapeDtypeStruct((N_NODES, D), jnp.float32),
        ],
    )(x, wa, wb)


def _gather_body(p_hbm, q_hbm, src_hbm, dst_hbm, g_hbm,
                 si_all, di_all, rs0, rs1, rs2, rs3, rs4,
                 rd0, rd1, rd2, rd3, rd4,
                 gsem_s, gsem_d, wsem):
    wid = lax.axis_index("s") * _NC + lax.axis_index("c")
    wbase = wid * _EPW
    rs = [rs0, rs1, rs2, rs3, rs4]
    rd = [rd0, rd1, rd2, rd3, rd4]

    def gather_descs(k, b):
        isl = pl.ds(k * _CH, _CH)
        return (
            pltpu.make_async_copy(p_hbm.at[si_all.at[isl]], rs[b], gsem_s.at[b]),
            pltpu.make_async_copy(q_hbm.at[di_all.at[isl]], rd[b], gsem_d.at[b]),
        )

    def wb_desc(k, b):
        osl = pl.ds(wbase + k * _CH, _CH)
        return pltpu.make_async_copy(rs[b], g_hbm.at[osl], wsem.at[b])

    pltpu.sync_copy(src_hbm.at[pl.ds(wbase, _EPW)], si_all)
    pltpu.sync_copy(dst_hbm.at[pl.ds(wbase, _EPW)], di_all)
    for dsc in gather_descs(0, 0):
        dsc.start()

    def outer(kk, carry):
        for b in range(_NB):
            k = kk * _NB + b
            k1 = k + 1
            nb = (b + 1) % _NB

            @pl.when(k1 < _NCHUNK)
            def _prefetch():
                @pl.when(k1 >= _NB)
                def _reclaim():
                    wb_desc(k1 - _NB, nb).wait()

                for dsc in gather_descs(k1, nb):
                    dsc.start()

            for dsc in gather_descs(k, b):
                dsc.wait()

            def row_body(r, c2):
                for j in range(D // 16):
                    sl = pl.ds(j * 16, 16)
                    rs[b][r, sl] = rs[b][r, sl] + rd[b][r, sl]
                return c2

            lax.fori_loop(0, _CH, row_body, 0)
            wb_desc(k, b).start()
        return carry

    lax.fori_loop(0, _NCHUNK // _NB, outer, 0)
    for b in range(_NB):
        wb_desc(_NCHUNK - _NB + b, b).wait()


def _gather_add(p, q, src, dst):
    mesh = plsc.VectorSubcoreMesh(core_axis_name="c", subcore_axis_name="s")
    fn = functools.partial(
        pl.kernel,
        mesh=mesh,
        out_type=jax.ShapeDtypeStruct((N_EDGES, D), jnp.float32),
        scratch_types=(
            [pltpu.VMEM((_EPW,), jnp.int32)] * 2
            + [pltpu.VMEM((_CH, D), jnp.float32)] * (2 * _NB)
            + [pltpu.SemaphoreType.DMA((_NB,))] * 3
        ),
    )(_gather_body)
    return fn(p, q, src, dst)


def _mlp_body(g_ref, ea_ref, w1c_ref, b1_ref, w2_ref, b2_ref, o_ref):
    h = g_ref[...] + jnp.dot(ea_ref[...], w1c_ref[...],
                             preferred_element_type=jnp.float32) + b1_ref[...]
    h = jnp.maximum(h, 0.0)
    o_ref[...] = jnp.dot(h, w2_ref[...],
                         preferred_element_type=jnp.float32) + b2_ref[...]


def _edge_mlp(g, ea, w1c, b1, w2, b2):
    blk = 2560
    grid = N_EDGES // blk
    de = ea.shape[1]
    eo = w2.shape[1]
    return pl.pallas_call(
        _mlp_body,
        grid=(grid,),
        in_specs=[
            pl.BlockSpec((blk, D), lambda i: (i, 0)),
            pl.BlockSpec((blk, de), lambda i: (i, 0)),
            pl.BlockSpec((de, D), lambda i: (0, 0)),
            pl.BlockSpec((1, D), lambda i: (0, 0)),
            pl.BlockSpec((D, eo), lambda i: (0, 0)),
            pl.BlockSpec((1, eo), lambda i: (0, 0)),
        ],
        out_specs=pl.BlockSpec((blk, eo), lambda i: (i, 0)),
        out_shape=jax.ShapeDtypeStruct((N_EDGES, eo), jnp.float32),
    )(g, ea, w1c, b1, w2, b2)


def kernel(x, edge_index, edge_attr, W1, b1, W2, b2):
    wa = W1[:D]
    wb = W1[D:2 * D]
    w1c = W1[2 * D:]
    p, q = _project_nodes(x, wa, wb)
    g = _gather_add(p, q, edge_index[0], edge_index[1])
    return _edge_mlp(g, edge_attr, w1c, b1.reshape(1, D), W2,
                     b2.reshape(1, -1))


# speedup vs baseline: 2.5837x; 1.5153x over previous
import functools

import jax
import jax.numpy as jnp
from jax import lax
from jax.experimental import pallas as pl
from jax.experimental.pallas import tpu as pltpu
from jax.experimental.pallas import tpu_sc as plsc

N_NODES = 10000
N_EDGES = 320000
D = 128

_NC = 2
_NS = 16
_NW = _NC * _NS
_EPW = N_EDGES // _NW
_CH = 80
_NCHUNK = _EPW // _CH
_NB = 5


def _proj_body(x_ref, wa_ref, wb_ref, p_ref, q_ref):
    xv = x_ref[...]
    p_ref[...] = jnp.dot(xv, wa_ref[...], preferred_element_type=jnp.float32)
    q_ref[...] = jnp.dot(xv, wb_ref[...], preferred_element_type=jnp.float32)


def _project_nodes(x, wa, wb):
    blk = 2000
    grid = N_NODES // blk
    return pl.pallas_call(
        _proj_body,
        grid=(grid,),
        in_specs=[
            pl.BlockSpec((blk, D), lambda i: (i, 0)),
            pl.BlockSpec((D, D), lambda i: (0, 0)),
            pl.BlockSpec((D, D), lambda i: (0, 0)),
        ],
        out_specs=[
            pl.BlockSpec((blk, D), lambda i: (i, 0)),
            pl.BlockSpec((blk, D), lambda i: (i, 0)),
        ],
        out_shape=[
            jax.ShapeDtypeStruct((N_NODES, D), jnp.float32),
            jax.ShapeDtypeStruct((N_NODES, D), jnp.float32),
        ],
    )(x, wa, wb)


def _gather_body(p_hbm, q_hbm, src_hbm, dst_hbm, g_hbm,
                 si_all, di_all, rs0, rs1, rs2, rs3, rs4,
                 rd0, rd1, rd2, rd3, rd4,
                 gsem_s, gsem_d, wsem):
    wid = lax.axis_index("s") * _NC + lax.axis_index("c")
    wbase = wid * _EPW
    rs = [rs0, rs1, rs2, rs3, rs4]
    rd = [rd0, rd1, rd2, rd3, rd4]

    def gather_descs(k, b):
        isl = pl.ds(k * _CH, _CH)
        return (
            pltpu.make_async_copy(p_hbm.at[si_all.at[isl]], rs[b], gsem_s.at[b]),
            pltpu.make_async_copy(q_hbm.at[di_all.at[isl]], rd[b], gsem_d.at[b]),
        )

    def wb_desc(k, b):
        osl = pl.ds(wbase + k * _CH, _CH)
        return pltpu.make_async_copy(rs[b], g_hbm.at[osl], wsem.at[b])

    pltpu.sync_copy(src_hbm.at[pl.ds(wbase, _EPW)], si_all)
    pltpu.sync_copy(dst_hbm.at[pl.ds(wbase, _EPW)], di_all)
    for dsc in gather_descs(0, 0):
        dsc.start()

    def outer(kk, carry):
        for b in range(_NB):
            k = kk * _NB + b
            k1 = k + 1
            nb = (b + 1) % _NB

            @pl.when(k1 < _NCHUNK)
            def _prefetch():
                @pl.when(k1 >= _NB)
                def _reclaim():
                    wb_desc(k1 - _NB, nb).wait()

                for dsc in gather_descs(k1, nb):
                    dsc.start()

            for dsc in gather_descs(k, b):
                dsc.wait()

            def row_body(r, c2):
                for j in range(D // 16):
                    sl = pl.ds(j * 16, 16)
                    rs[b][r, sl] = rs[b][r, sl] + rd[b][r, sl]
                return c2

            lax.fori_loop(0, _CH, row_body, 0)
            wb_desc(k, b).start()
        return carry

    lax.fori_loop(0, _NCHUNK // _NB, outer, 0)
    for b in range(_NB):
        wb_desc(_NCHUNK - _NB + b, b).wait()


def _gather_add(p, q, src, dst):
    mesh = plsc.VectorSubcoreMesh(core_axis_name="c", subcore_axis_name="s")
    fn = functools.partial(
        pl.kernel,
        mesh=mesh,
        out_type=jax.ShapeDtypeStruct((N_EDGES, D), jnp.float32),
        scratch_types=(
            [pltpu.VMEM((_EPW,), jnp.int32)] * 2
            + [pltpu.VMEM((_CH, D), jnp.float32)] * (2 * _NB)
            + [pltpu.SemaphoreType.DMA((_NB,))] * 3
        ),
    )(_gather_body)
    return fn(p, q, src, dst)


def _mlp_body(g_ref, eat_ref, w1c_ref, b1_ref, w2_ref, b2_ref, ot_ref):
    c = lax.dot_general(eat_ref[...], w1c_ref[...],
                        (((0,), (0,)), ((), ())),
                        preferred_element_type=jnp.float32)
    h = g_ref[...] + c + b1_ref[...]
    h = jnp.maximum(h, 0.0)
    ot_ref[...] = lax.dot_general(w2_ref[...], h,
                                  (((0,), (1,)), ((), ())),
                                  preferred_element_type=jnp.float32
                                  ) + b2_ref[...]


def _edge_mlp(g, ea_t, w1c, b1, w2, b2):
    blk = 2560
    grid = N_EDGES // blk
    de = ea_t.shape[0]
    eo = w2.shape[1]
    out_t = pl.pallas_call(
        _mlp_body,
        grid=(grid,),
        in_specs=[
            pl.BlockSpec((blk, D), lambda i: (i, 0)),
            pl.BlockSpec((de, blk), lambda i: (0, i)),
            pl.BlockSpec((de, D), lambda i: (0, 0)),
            pl.BlockSpec((1, D), lambda i: (0, 0)),
            pl.BlockSpec((D, eo), lambda i: (0, 0)),
            pl.BlockSpec((eo, 1), lambda i: (0, 0)),
        ],
        out_specs=pl.BlockSpec((eo, blk), lambda i: (0, i)),
        out_shape=jax.ShapeDtypeStruct((eo, N_EDGES), jnp.float32),
    )(g, ea_t, w1c, b1, w2, b2)
    return out_t.T


def kernel(x, edge_index, edge_attr, W1, b1, W2, b2):
    wa = W1[:D]
    wb = W1[D:2 * D]
    w1c = W1[2 * D:]
    p, q = _project_nodes(x, wa, wb)
    g = _gather_add(p, q, edge_index[0], edge_index[1])
    return _edge_mlp(g, edge_attr.T, w1c, b1.reshape(1, D), W2,
                     b2.reshape(-1, 1))


# speedup vs baseline: 2.9248x; 1.1320x over previous
import functools

import jax
import jax.numpy as jnp
from jax import lax
from jax.experimental import pallas as pl
from jax.experimental.pallas import tpu as pltpu
from jax.experimental.pallas import tpu_sc as plsc

N_NODES = 10000
N_EDGES = 320000
D = 128

_NC = 2
_NS = 16
_NW = _NC * _NS
_NSLICE = 5
_ES = N_EDGES // _NSLICE
_EPW = _ES // _NW
_CH = 80
_NCHUNK = _EPW // _CH
_NB = 5


def _proj_body(x_ref, wa_ref, wb_ref, p_ref, q_ref):
    xv = x_ref[...]
    p_ref[...] = jnp.dot(xv, wa_ref[...], preferred_element_type=jnp.float32)
    q_ref[...] = jnp.dot(xv, wb_ref[...], preferred_element_type=jnp.float32)


def _project_nodes(x, wa, wb):
    blk = 2000
    grid = N_NODES // blk
    return pl.pallas_call(
        _proj_body,
        grid=(grid,),
        in_specs=[
            pl.BlockSpec((blk, D), lambda i: (i, 0)),
            pl.BlockSpec((D, D), lambda i: (0, 0)),
            pl.BlockSpec((D, D), lambda i: (0, 0)),
        ],
        out_specs=[
            pl.BlockSpec((blk, D), lambda i: (i, 0)),
            pl.BlockSpec((blk, D), lambda i: (i, 0)),
        ],
        out_shape=[
            jax.ShapeDtypeStruct((N_NODES, D), jnp.float32),
            jax.ShapeDtypeStruct((N_NODES, D), jnp.float32),
        ],
    )(x, wa, wb)


def _gather_body(p_hbm, q_hbm, src_hbm, dst_hbm, g_hbm,
                 si_all, di_all, rs0, rs1, rs2, rs3, rs4,
                 rd0, rd1, rd2, rd3, rd4,
                 gsem_s, gsem_d, wsem):
    wid = lax.axis_index("s") * _NC + lax.axis_index("c")
    wbase = wid * _EPW
    rs = [rs0, rs1, rs2, rs3, rs4]
    rd = [rd0, rd1, rd2, rd3, rd4]

    def gather_descs(k, b):
        isl = pl.ds(k * _CH, _CH)
        return (
            pltpu.make_async_copy(p_hbm.at[si_all.at[isl]], rs[b], gsem_s.at[b]),
            pltpu.make_async_copy(q_hbm.at[di_all.at[isl]], rd[b], gsem_d.at[b]),
        )

    def wb_desc(k, b):
        osl = pl.ds(wbase + k * _CH, _CH)
        return pltpu.make_async_copy(rs[b], g_hbm.at[osl], wsem.at[b])

    pltpu.sync_copy(src_hbm.at[pl.ds(wbase, _EPW)], si_all)
    pltpu.sync_copy(dst_hbm.at[pl.ds(wbase, _EPW)], di_all)
    for dsc in gather_descs(0, 0):
        dsc.start()

    def outer(kk, carry):
        for b in range(_NB):
            k = kk * _NB + b
            k1 = k + 1
            nb = (b + 1) % _NB

            @pl.when(k1 < _NCHUNK)
            def _prefetch():
                @pl.when(k1 >= _NB)
                def _reclaim():
                    wb_desc(k1 - _NB, nb).wait()

                for dsc in gather_descs(k1, nb):
                    dsc.start()

            for dsc in gather_descs(k, b):
                dsc.wait()

            def row_body(r, c2):
                for j in range(D // 16):
                    sl = pl.ds(j * 16, 16)
                    rs[b][r, sl] = rs[b][r, sl] + rd[b][r, sl]
                return c2

            lax.fori_loop(0, _CH, row_body, 0)
            wb_desc(k, b).start()
        return carry

    lax.fori_loop(0, _NCHUNK // _NB, outer, 0)
    for b in range(_NB):
        wb_desc(_NCHUNK - _NB + b, b).wait()


def _gather_add(p, q, src, dst):
    mesh = plsc.VectorSubcoreMesh(core_axis_name="c", subcore_axis_name="s")
    fn = functools.partial(
        pl.kernel,
        mesh=mesh,
        out_type=jax.ShapeDtypeStruct((_ES, D), jnp.float32),
        scratch_types=(
            [pltpu.VMEM((_EPW,), jnp.int32)] * 2
            + [pltpu.VMEM((_CH, D), jnp.float32)] * (2 * _NB)
            + [pltpu.SemaphoreType.DMA((_NB,))] * 3
        ),
    )(_gather_body)
    return fn(p, q, src, dst)


def _mlp_body(g_ref, eat_ref, w1c_ref, b1_ref, w2_ref, b2_ref, ot_ref):
    c = lax.dot_general(eat_ref[...], w1c_ref[...],
                        (((0,), (0,)), ((), ())),
                        preferred_element_type=jnp.float32)
    h = g_ref[...] + c + b1_ref[...]
    h = jnp.maximum(h, 0.0)
    ot_ref[...] = lax.dot_general(w2_ref[...], h,
                                  (((0,), (1,)), ((), ())),
                                  preferred_element_type=jnp.float32
                                  ) + b2_ref[...]


def _edge_mlp(g, ea_t, w1c, b1, w2, b2):
    blk = 2560
    grid = _ES // blk
    de = ea_t.shape[0]
    eo = w2.shape[1]
    out_t = pl.pallas_call(
        _mlp_body,
        grid=(grid,),
        in_specs=[
            pl.BlockSpec((blk, D), lambda i: (i, 0)),
            pl.BlockSpec((de, blk), lambda i: (0, i)),
            pl.BlockSpec((de, D), lambda i: (0, 0)),
            pl.BlockSpec((1, D), lambda i: (0, 0)),
            pl.BlockSpec((D, eo), lambda i: (0, 0)),
            pl.BlockSpec((eo, 1), lambda i: (0, 0)),
        ],
        out_specs=pl.BlockSpec((eo, blk), lambda i: (0, i)),
        out_shape=jax.ShapeDtypeStruct((eo, _ES), jnp.float32),
    )(g, ea_t, w1c, b1, w2, b2)
    return out_t


def kernel(x, edge_index, edge_attr, W1, b1, W2, b2):
    wa = W1[:D]
    wb = W1[D:2 * D]
    w1c = W1[2 * D:]
    p, q = _project_nodes(x, wa, wb)
    src = edge_index[0]
    dst = edge_index[1]
    ea_t = edge_attr.T
    b1r = b1.reshape(1, D)
    b2r = b2.reshape(-1, 1)
    outs = []
    for i in range(_NSLICE):
        lo = i * _ES
        g = _gather_add(p, q, lax.dynamic_slice_in_dim(src, lo, _ES),
                        lax.dynamic_slice_in_dim(dst, lo, _ES))
        outs.append(_edge_mlp(g, lax.dynamic_slice_in_dim(ea_t, lo, _ES, 1),
                              w1c, b1r, W2, b2r))
    return jnp.concatenate(outs, axis=1).T
